# Initial kernel scaffold; baseline (speedup 1.0000x reference)
import jax
import jax.numpy as jnp
from jax.experimental import pallas as pl


def kernel(y, wids, lora_B):
    raise NotImplementedError("write your pallas kernel here")


# speedup vs baseline: 1.8470x; 1.8470x over previous
import functools

import jax
import jax.numpy as jnp
from jax import lax
from jax.experimental import pallas as pl
from jax.experimental.pallas import tpu as pltpu
from jax.experimental.pallas import tpu_sc as plsc

LORA_BATCH = 320
R_SMALL = 16
NUM_ADAPTERS = 80
D_OUT = 4096
N_OUT_ROWS = 128
K_DIM = NUM_ADAPTERS * R_SMALL
PAD_ROWS = 384

_sc_mesh = plsc.VectorSubcoreMesh(core_axis_name="c", subcore_axis_name="s")


@functools.partial(
    pl.kernel,
    mesh=_sc_mesh,
    out_type=jax.ShapeDtypeStruct((N_OUT_ROWS, K_DIM), jnp.float32),
    scratch_types=[
        pltpu.VMEM((16, R_SMALL), jnp.float32),
        pltpu.VMEM((16,), jnp.int32),
        pltpu.VMEM((8, K_DIM), jnp.float32),
    ],
    compiler_params=pltpu.CompilerParams(needs_layout_passes=False),
)
def _sc_scatter(y_hbm, wids_hbm, s_hbm, yv, wv, acc):
    w = lax.axis_index("s") * 2 + lax.axis_index("c")
    is_large = w < 16
    is_small = jnp.logical_and(w >= 16, w < 24)
    ws = jnp.minimum(w - 16, 7)
    ibase = jnp.where(is_large, 16 * w, 256 + 8 * ws)
    nitems = jnp.where(is_large, 16, jnp.where(w < 24, 8, 0))

    pltpu.sync_copy(y_hbm.at[pl.ds(ibase, 16)], yv)
    pltpu.sync_copy(wids_hbm.at[pl.ds(ibase, 16)], wv)

    zeros16 = jnp.zeros((R_SMALL,), jnp.float32)

    def _zero_body(i, carry):
        r = i // (K_DIM // R_SMALL)
        cchunk = i % (K_DIM // R_SMALL)
        acc[r, pl.ds(cchunk * R_SMALL, R_SMALL)] = zeros16
        return carry

    lax.fori_loop(0, 8 * (K_DIM // R_SMALL), _zero_body, 0)

    lane_iota = lax.iota(jnp.int32, 16)
    wvec = wv[...]
    for j in range(16):
        @pl.when(j < nitems)
        def _():
            wid_j = wvec[j]
            yj = yv[j, :]
            r = jnp.where(is_large, j // 4, j)
            rvec = lax.broadcast(r, (16,))
            col = lax.broadcast(wid_j * R_SMALL, (16,)) + lane_iota
            plsc.addupdate_scatter(acc, [rvec, col], yj)

    @pl.when(is_large)
    def _():
        pltpu.sync_copy(acc.at[pl.ds(0, 4)], s_hbm.at[pl.ds(4 * w, 4)])

    @pl.when(is_small)
    def _():
        pltpu.sync_copy(acc.at[pl.ds(0, 8)], s_hbm.at[pl.ds(64 + 8 * ws, 8)])


def _mm_body(s_ref, b_ref, o_ref):
    acc = jnp.dot(
        s_ref[...],
        b_ref[...],
        preferred_element_type=jnp.float32,
    )
    o_ref[...] = acc * 2.0


_N_BLK = 512


def _tc_matmul(s, b_flat):
    return pl.pallas_call(
        _mm_body,
        grid=(D_OUT // _N_BLK,),
        in_specs=[
            pl.BlockSpec((N_OUT_ROWS, K_DIM), lambda i: (0, 0)),
            pl.BlockSpec((K_DIM, _N_BLK), lambda i: (0, i)),
        ],
        out_specs=pl.BlockSpec((N_OUT_ROWS, _N_BLK), lambda i: (0, i)),
        out_shape=jax.ShapeDtypeStruct((N_OUT_ROWS, D_OUT), jnp.float32),
        compiler_params=pltpu.CompilerParams(
            allow_input_fusion=[True, True],
        ),
    )(s, b_flat)


@jax.jit
def kernel(y, wids, lora_B):
    y32 = y[:, 0, :].astype(jnp.float32)
    y32p = jnp.pad(y32, ((0, PAD_ROWS - LORA_BATCH), (0, 0)))
    widsp = jnp.pad(wids, (0, PAD_ROWS - LORA_BATCH))
    s = _sc_scatter(y32p, widsp)
    out = _tc_matmul(
        s.astype(jnp.bfloat16),
        lora_B.reshape(K_DIM, D_OUT).astype(jnp.bfloat16),
    )
    return out.astype(jnp.float16).reshape(N_OUT_ROWS, 1, D_OUT)


# speedup vs baseline: 1.8564x; 1.0051x over previous
import functools

import jax
import jax.numpy as jnp
from jax import lax
from jax.experimental import pallas as pl
from jax.experimental.pallas import tpu as pltpu
from jax.experimental.pallas import tpu_sc as plsc

LORA_BATCH = 320
R_SMALL = 16
NUM_ADAPTERS = 80
D_OUT = 4096
N_OUT_ROWS = 128
K_DIM = NUM_ADAPTERS * R_SMALL
PAD_ROWS = 384

_sc_mesh = plsc.VectorSubcoreMesh(core_axis_name="c", subcore_axis_name="s")


@functools.partial(
    pl.kernel,
    mesh=_sc_mesh,
    out_type=jax.ShapeDtypeStruct((N_OUT_ROWS, K_DIM), jnp.float32),
    scratch_types=[
        pltpu.VMEM((16, R_SMALL), jnp.float32),
        pltpu.VMEM((16,), jnp.int32),
        pltpu.VMEM((8, K_DIM), jnp.float32),
    ],
    compiler_params=pltpu.CompilerParams(needs_layout_passes=False),
)
def _sc_scatter(y_hbm, wids_hbm, s_hbm, yv, wv, acc):
    w = lax.axis_index("s") * 2 + lax.axis_index("c")
    is_large = w < 16
    is_small = jnp.logical_and(w >= 16, w < 24)
    ws = jnp.minimum(w - 16, 7)
    ibase = jnp.where(is_large, 16 * w, 256 + 8 * ws)
    nitems = jnp.where(is_large, 16, jnp.where(w < 24, 8, 0))

    pltpu.sync_copy(y_hbm.at[pl.ds(ibase, 16)], yv)
    pltpu.sync_copy(wids_hbm.at[pl.ds(ibase, 16)], wv)

    zeros16 = jnp.zeros((R_SMALL,), jnp.float32)

    def _zero_body(i, carry):
        r = i // (K_DIM // R_SMALL)
        cchunk = i % (K_DIM // R_SMALL)
        acc[r, pl.ds(cchunk * R_SMALL, R_SMALL)] = zeros16
        return carry

    lax.fori_loop(0, 8 * (K_DIM // R_SMALL), _zero_body, 0)

    lane_iota = lax.iota(jnp.int32, 16)
    wvec = wv[...]
    for j in range(16):
        @pl.when(j < nitems)
        def _():
            wid_j = wvec[j]
            yj = yv[j, :]
            r = jnp.where(is_large, j // 4, j)
            rvec = lax.broadcast(r, (16,))
            col = lax.broadcast(wid_j * R_SMALL, (16,)) + lane_iota
            plsc.addupdate_scatter(acc, [rvec, col], yj)

    @pl.when(is_large)
    def _():
        pltpu.sync_copy(acc.at[pl.ds(0, 4)], s_hbm.at[pl.ds(4 * w, 4)])

    @pl.when(is_small)
    def _():
        pltpu.sync_copy(acc.at[pl.ds(0, 8)], s_hbm.at[pl.ds(64 + 8 * ws, 8)])


def _mm_body(s_ref, b_ref, o_ref):
    acc = jnp.dot(
        s_ref[...],
        b_ref[...],
        preferred_element_type=jnp.float32,
    )
    o_ref[...] = (acc * 2.0).astype(jnp.bfloat16)


_N_BLK = 512


def _tc_matmul(s, b_flat):
    return pl.pallas_call(
        _mm_body,
        grid=(D_OUT // _N_BLK,),
        in_specs=[
            pl.BlockSpec((N_OUT_ROWS, K_DIM), lambda i: (0, 0)),
            pl.BlockSpec((K_DIM, _N_BLK), lambda i: (0, i)),
        ],
        out_specs=pl.BlockSpec((N_OUT_ROWS, _N_BLK), lambda i: (0, i)),
        out_shape=jax.ShapeDtypeStruct((N_OUT_ROWS, D_OUT), jnp.bfloat16),
        compiler_params=pltpu.CompilerParams(
            allow_input_fusion=[True, True],
        ),
    )(s, b_flat)


@jax.jit
def kernel(y, wids, lora_B):
    y32 = y[:, 0, :].astype(jnp.float32)
    y32p = jnp.pad(y32, ((0, PAD_ROWS - LORA_BATCH), (0, 0)))
    widsp = jnp.pad(wids, (0, PAD_ROWS - LORA_BATCH))
    s = _sc_scatter(y32p, widsp)
    out = _tc_matmul(
        s.astype(jnp.bfloat16),
        lora_B.reshape(K_DIM, D_OUT).astype(jnp.bfloat16),
    )
    return out.astype(jnp.float16).reshape(N_OUT_ROWS, 1, D_OUT)
